# baseline (speedup 1.0000x reference)
import functools

import jax
import jax.numpy as jnp
from jax import lax
from jax.experimental import pallas as pl
from jax.experimental.pallas import tpu as pltpu
from jax.experimental.pallas import tpu_sc as plsc

NUM_EXPERTS = 64
TOP_K = 8
HIDDEN = 4096
TOKENS = 16384
BLOCK_T = 1024

_INFO = plsc.get_sparse_core_info()
NC, NS, L = _INFO.num_cores, _INFO.num_subcores, _INFO.num_lanes
NW = NC * NS
TPW = TOKENS // NW
GROUPS = TPW // L


def _gating_block(x_ref, w_ref, logits_ref, probs_t_ref):
    x = x_ref[...]
    w = w_ref[...]
    logits_t = jax.lax.dot_general(
        w, x, (((1,), (1,)), ((), ())), preferred_element_type=jnp.float32
    )
    logits_ref[...] = logits_t.T
    m = jnp.max(logits_t, axis=0, keepdims=True)
    e = jnp.exp(logits_t - m)
    probs_t_ref[...] = e / jnp.sum(e, axis=0, keepdims=True)


def _gating(input, weight):
    grid = (TOKENS // BLOCK_T,)
    return pl.pallas_call(
        _gating_block,
        grid=grid,
        in_specs=[
            pl.BlockSpec((BLOCK_T, HIDDEN), lambda i: (i, 0)),
            pl.BlockSpec((NUM_EXPERTS, HIDDEN), lambda i: (0, 0)),
        ],
        out_specs=[
            pl.BlockSpec((BLOCK_T, NUM_EXPERTS), lambda i: (i, 0)),
            pl.BlockSpec((NUM_EXPERTS, BLOCK_T), lambda i: (0, i)),
        ],
        out_shape=[
            jax.ShapeDtypeStruct((TOKENS, NUM_EXPERTS), jnp.float32),
            jax.ShapeDtypeStruct((NUM_EXPERTS, TOKENS), jnp.float32),
        ],
    )(input, weight)


@functools.partial(
    pl.kernel,
    out_type=[
        jax.ShapeDtypeStruct((TOP_K, TOKENS), jnp.float32),
        jax.ShapeDtypeStruct((TOP_K, TOKENS), jnp.int32),
    ],
    mesh=plsc.VectorSubcoreMesh(core_axis_name="c", subcore_axis_name="s"),
    scratch_types=[
        pltpu.VMEM((NUM_EXPERTS, TPW), jnp.float32),
        pltpu.VMEM((TOP_K, TPW), jnp.float32),
        pltpu.VMEM((TOP_K, TPW), jnp.int32),
    ],
)
def _topk_sc(probs_t_hbm, scores_hbm, idx_hbm, probs_v, out_s, out_i):
    wid = lax.axis_index("s") * NC + lax.axis_index("c")
    base = wid * TPW
    pltpu.sync_copy(probs_t_hbm.at[:, pl.ds(base, TPW)], probs_v)

    lane = lax.broadcasted_iota(jnp.int32, (L,), 0)

    def per_group(g, _):
        vals = [jnp.full((L,), -1.0, jnp.float32) for _ in range(TOP_K)]
        idxs = [jnp.full((L,), 0, jnp.int32) for _ in range(TOP_K)]
        col = g * L
        for e in range(NUM_EXPERTS):
            v = probs_v[e, pl.ds(col, L)]
            i = jnp.full((L,), e, jnp.int32)
            for j in range(TOP_K):
                c = v > vals[j]
                vals[j], v = (
                    jnp.where(c, v, vals[j]),
                    jnp.where(c, vals[j], v),
                )
                idxs[j], i = (
                    jnp.where(c, i, idxs[j]),
                    jnp.where(c, idxs[j], i),
                )
        for j in range(TOP_K):
            out_s[j, pl.ds(col, L)] = vals[j]
            out_i[j, pl.ds(col, L)] = idxs[j]
        return _

    lax.fori_loop(0, GROUPS, per_group, 0)
    pltpu.sync_copy(out_s, scores_hbm.at[:, pl.ds(base, TPW)])
    pltpu.sync_copy(out_i, idx_hbm.at[:, pl.ds(base, TPW)])


@jax.jit
def kernel(input, weight):
    logits, probs_t = _gating(input, weight)
    top_scores, top_indices = _topk_sc(probs_t)
    return top_scores.T, top_indices.T, logits
